# Initial kernel scaffold; baseline (speedup 1.0000x reference)
import jax
import jax.numpy as jnp
from jax.experimental import pallas as pl


def kernel(X, edge_index, W1, b1, W2, b2, W3, b3, W4, b4):
    raise NotImplementedError("write your pallas kernel here")


# speedup vs baseline: 4.5121x; 4.5121x over previous
import jax
import jax.numpy as jnp
from jax import lax
from jax.experimental import pallas as pl
from jax.experimental.pallas import tpu as pltpu
from jax.experimental.pallas import tpu_sc as plsc

N = 10000
E = 320000
D = 128
H1 = 64
H2 = 128
HFC = 256

NC = 2
NS = 16
LANES = 16
NW = NC * NS
EPW = E // NW
CH = 80
NCH = EPW // CH
RPT = N // NS

BLK = 400


def _make_sc_edge(h, with_count):
    mesh = plsc.VectorSubcoreMesh(core_axis_name="c", subcore_axis_name="s")
    out_type = [jax.ShapeDtypeStruct((NC, N, h), jnp.float32)]
    scratch = [
        pltpu.VMEM((NCH, CH), jnp.int32),
        pltpu.VMEM((NCH, CH), jnp.int32),
    ]
    scratch += [
        pltpu.VMEM((CH, h), jnp.float32),
        pltpu.VMEM((CH, h), jnp.float32),
        pltpu.VMEM((CH, h), jnp.float32),
        pltpu.VMEM_SHARED((N, h), jnp.float32),
        pltpu.SemaphoreType.DMA,
        pltpu.SemaphoreType.DMA,
    ]
    if with_count:
        out_type.append(jax.ShapeDtypeStruct((NC, N, LANES), jnp.float32))
        scratch += [
            pltpu.VMEM((CH, LANES), jnp.float32),
            pltpu.VMEM_SHARED((N, LANES), jnp.float32),
        ]

    def body(*refs):
        if with_count:
            (a_hbm, b_hbm, dst_hbm, src_hbm, zsum_hbm, zcnt_hbm,
             out_sum, out_cnt,
             dst_v, src_v, a_v, b_v, m_v, acc_s, sem_a, sem_b,
             ones_v, cnt_s) = refs
        else:
            (a_hbm, b_hbm, dst_hbm, src_hbm, zsum_hbm,
             out_sum,
             dst_v, src_v, a_v, b_v, m_v, acc_s, sem_a, sem_b) = refs
        cid = lax.axis_index("c")
        sid = lax.axis_index("s")
        wid = cid * NS + sid
        r0 = sid * RPT
        pltpu.sync_copy(zsum_hbm.at[pl.ds(r0, RPT)], acc_s.at[pl.ds(r0, RPT)])
        if with_count:
            pltpu.sync_copy(zcnt_hbm.at[pl.ds(r0, RPT)],
                            cnt_s.at[pl.ds(r0, RPT)])
            ones = jnp.full((LANES,), 1.0, jnp.float32)

            def fill(e, _):
                ones_v[e, :] = ones
                return 0

            lax.fori_loop(0, CH, fill, 0)
        pltpu.sync_copy(dst_hbm.at[wid], dst_v)
        pltpu.sync_copy(src_hbm.at[wid], src_v)
        plsc.subcore_barrier()

        def chunk(c, _):
            di = dst_v.at[c]
            si = src_v.at[c]
            ga = pltpu.async_copy(a_hbm.at[di], a_v, sem_a)
            gb = pltpu.async_copy(b_hbm.at[si], b_v, sem_b)
            ga.wait()
            gb.wait()

            def edge(e, _):
                for k in range(h // LANES):
                    s = k * LANES
                    m_v[e, pl.ds(s, LANES)] = jnp.maximum(
                        a_v[e, pl.ds(s, LANES)] + b_v[e, pl.ds(s, LANES)],
                        0.0)
                return 0

            lax.fori_loop(0, CH, edge, 0, unroll=2)
            pltpu.sync_copy(m_v, acc_s.at[di], add=True)
            if with_count:
                pltpu.sync_copy(ones_v, cnt_s.at[di], add=True)
            return 0

        lax.fori_loop(0, NCH, chunk, 0)
        plsc.subcore_barrier()
        pltpu.sync_copy(acc_s.at[pl.ds(r0, RPT)],
                        out_sum.at[cid, pl.ds(r0, RPT)])
        if with_count:
            pltpu.sync_copy(cnt_s.at[pl.ds(r0, RPT)],
                            out_cnt.at[cid, pl.ds(r0, RPT)])

    return pl.kernel(body, out_type=out_type, mesh=mesh,
                     scratch_types=scratch,
                     compiler_params=pltpu.CompilerParams(
                         use_tc_tiling_on_sc=False))


def _tc1_body(x_ref, w_ref, b_ref, a_ref, bo_ref):
    x = x_ref[...]
    w = w_ref[...]
    wa = w[:, :D]
    wb = w[:, D:]
    dn = (((1,), (1,)), ((), ()))
    a_ref[...] = lax.dot_general(x, wa - wb, dn) + b_ref[...]
    bo_ref[...] = lax.dot_general(x, wb, dn)


def _tc2_body(p_ref, c_ref, w_ref, b_ref, a_ref, bo_ref):
    s = p_ref[0] + p_ref[1]
    cnt = c_ref[0, :, 0] + c_ref[1, :, 0]
    hn = s / jnp.maximum(cnt, 1.0)[:, None]
    w = w_ref[...]
    wa = w[:, :H1]
    wb = w[:, H1:]
    dn = (((1,), (1,)), ((), ()))
    a_ref[...] = lax.dot_general(hn, wa - wb, dn) + b_ref[...]
    bo_ref[...] = lax.dot_general(hn, wb, dn)


def _tc3_body(p_ref, c_ref, w3_ref, b3_ref, w4_ref, b4_ref, o_ref):
    s = p_ref[0] + p_ref[1]
    cnt = c_ref[0, :, 0] + c_ref[1, :, 0]
    hn = s / jnp.maximum(cnt, 1.0)[:, None]
    dn = (((1,), (1,)), ((), ()))
    f = jnp.maximum(lax.dot_general(hn, w3_ref[...], dn) + b3_ref[...], 0.0)
    o = jnp.sum(f * w4_ref[...], axis=1, keepdims=True) + b4_ref[0, 0]
    o_ref[...] = jax.nn.sigmoid(o)


def kernel(X, edge_index, W1, b1, W2, b2, W3, b3, W4, b4):
    f32 = jnp.float32
    src2d = edge_index[0].reshape(NW, NCH, CH)
    dst2d = edge_index[1].reshape(NW, NCH, CH)
    zs1 = jnp.zeros((N, H1), f32)
    zs2 = jnp.zeros((N, H2), f32)
    zc = jnp.zeros((N, LANES), f32)

    grid = (N // BLK,)
    full = lambda shape: pl.BlockSpec(shape, lambda i: (0,) * len(shape))
    rows = lambda w: pl.BlockSpec((BLK, w), lambda i: (i, 0))
    parts = lambda w: pl.BlockSpec((NC, BLK, w), lambda i: (0, i, 0))

    A1, B1 = pl.pallas_call(
        _tc1_body,
        grid=grid,
        in_specs=[rows(D), full((H1, 2 * D)), full((1, H1))],
        out_specs=[rows(H1), rows(H1)],
        out_shape=[jax.ShapeDtypeStruct((N, H1), f32)] * 2,
    )(X, W1, b1.reshape(1, H1))

    P1, CNT = _make_sc_edge(H1, True)(A1, B1, dst2d, src2d, zs1, zc)

    A2, B2 = pl.pallas_call(
        _tc2_body,
        grid=grid,
        in_specs=[parts(H1), parts(LANES), full((H2, 2 * H1)),
                  full((1, H2))],
        out_specs=[rows(H2), rows(H2)],
        out_shape=[jax.ShapeDtypeStruct((N, H2), f32)] * 2,
    )(P1, CNT, W2, b2.reshape(1, H2))

    (P2,) = _make_sc_edge(H2, False)(A2, B2, dst2d, src2d, zs2)

    out = pl.pallas_call(
        _tc3_body,
        grid=grid,
        in_specs=[parts(H2), parts(LANES), full((HFC, H2)), full((1, HFC)),
                  full((1, HFC)), full((1, 1))],
        out_specs=[rows(1)],
        out_shape=[jax.ShapeDtypeStruct((N, 1), f32)],
    )(P2, CNT, W3, b3.reshape(1, HFC), W4, b4.reshape(1, 1))[0]

    return out.reshape(N)


# speedup vs baseline: 6.5215x; 1.4454x over previous
import jax
import jax.numpy as jnp
from jax import lax
from jax.experimental import pallas as pl
from jax.experimental.pallas import tpu as pltpu
from jax.experimental.pallas import tpu_sc as plsc

N = 10000
E = 320000
D = 128
H1 = 64
H2 = 128
HFC = 256

NC = 2
NS = 16
LANES = 16
NW = NC * NS
EPW = E // NW
RPT = N // NS

BLK = 400


def _make_sc_edge(h, with_count, CH):
    NCH = EPW // CH
    mesh = plsc.VectorSubcoreMesh(core_axis_name="c", subcore_axis_name="s")
    out_type = [jax.ShapeDtypeStruct((NC, N, h), jnp.float32)]
    scratch = [
        pltpu.VMEM((NCH, CH), jnp.int32),
        pltpu.VMEM((NCH, CH), jnp.int32),
    ]
    scratch += [pltpu.VMEM((CH, h), jnp.float32)] * 6
    scratch += [pltpu.VMEM_SHARED((N, h), jnp.float32)]
    scratch += [pltpu.SemaphoreType.DMA] * 6
    if with_count:
        out_type.append(jax.ShapeDtypeStruct((NC, N, LANES), jnp.float32))
        scratch += [
            pltpu.VMEM((CH, LANES), jnp.float32),
            pltpu.VMEM_SHARED((N, LANES), jnp.float32),
            pltpu.SemaphoreType.DMA,
            pltpu.SemaphoreType.DMA,
        ]

    def body(*refs):
        if with_count:
            (a_hbm, b_hbm, dst_hbm, src_hbm, zsum_hbm, zcnt_hbm,
             out_sum, out_cnt,
             dst_v, src_v, a0, a1, b0, b1, m0, m1, acc_s,
             sa0, sa1, sb0, sb1, ss0, ss1,
             ones_v, cnt_s, scc0, scc1) = refs
        else:
            (a_hbm, b_hbm, dst_hbm, src_hbm, zsum_hbm,
             out_sum,
             dst_v, src_v, a0, a1, b0, b1, m0, m1, acc_s,
             sa0, sa1, sb0, sb1, ss0, ss1) = refs
            ones_v = cnt_s = scc0 = scc1 = None
        cid = lax.axis_index("c")
        sid = lax.axis_index("s")
        wid = cid * NS + sid
        r0 = sid * RPT
        pltpu.sync_copy(zsum_hbm.at[pl.ds(r0, RPT)], acc_s.at[pl.ds(r0, RPT)])
        if with_count:
            pltpu.sync_copy(zcnt_hbm.at[pl.ds(r0, RPT)],
                            cnt_s.at[pl.ds(r0, RPT)])
            ones = jnp.full((LANES,), 1.0, jnp.float32)

            def fill(e, _):
                ones_v[e, :] = ones
                return 0

            lax.fori_loop(0, CH, fill, 0)
        pltpu.sync_copy(dst_hbm.at[wid], dst_v)
        pltpu.sync_copy(src_hbm.at[wid], src_v)
        plsc.subcore_barrier()

        def gather_pair(c, av, bv, sa, sb):
            pltpu.async_copy(a_hbm.at[dst_v.at[c]], av, sa)
            pltpu.async_copy(b_hbm.at[src_v.at[c]], bv, sb)

        def wait_gather(av, bv, sa, sb):
            pltpu.make_async_copy(a_hbm.at[dst_v.at[0]], av, sa).wait()
            pltpu.make_async_copy(b_hbm.at[src_v.at[0]], bv, sb).wait()

        def compute(av, bv, mv):
            def edge(e, _):
                for k in range(h // LANES):
                    s = k * LANES
                    mv[e, pl.ds(s, LANES)] = jnp.maximum(
                        av[e, pl.ds(s, LANES)] + bv[e, pl.ds(s, LANES)],
                        0.0)
                return 0

            lax.fori_loop(0, CH, edge, 0, unroll=4)

        def scatter(c, mv, ss, scc):
            pltpu.async_copy(mv, acc_s.at[dst_v.at[c]], ss, add=True)
            if with_count:
                pltpu.async_copy(ones_v, cnt_s.at[dst_v.at[c]], scc,
                                 add=True)

        def wait_scatter(mv, ss, scc):
            pltpu.make_async_copy(mv, acc_s.at[dst_v.at[0]], ss).wait()
            if with_count:
                pltpu.make_async_copy(ones_v, cnt_s.at[dst_v.at[0]],
                                      scc).wait()

        gather_pair(0, a0, b0, sa0, sb0)

        def it(i, _):
            c0 = 2 * i
            gather_pair(c0 + 1, a1, b1, sa1, sb1)
            wait_gather(a0, b0, sa0, sb0)

            @pl.when(i > 0)
            def _():
                wait_scatter(m0, ss0, scc0)

            compute(a0, b0, m0)
            scatter(c0, m0, ss0, scc0)
            gather_pair(c0 + 2, a0, b0, sa0, sb0)
            wait_gather(a1, b1, sa1, sb1)

            @pl.when(i > 0)
            def _():
                wait_scatter(m1, ss1, scc1)

            compute(a1, b1, m1)
            scatter(c0 + 1, m1, ss1, scc1)
            return 0

        lax.fori_loop(0, (NCH - 1) // 2, it, 0)
        wait_gather(a0, b0, sa0, sb0)
        wait_scatter(m0, ss0, scc0)
        compute(a0, b0, m0)
        scatter(NCH - 1, m0, ss0, scc0)
        wait_scatter(m0, ss0, scc0)
        wait_scatter(m1, ss1, scc1)
        plsc.subcore_barrier()
        pltpu.sync_copy(acc_s.at[pl.ds(r0, RPT)],
                        out_sum.at[cid, pl.ds(r0, RPT)])
        if with_count:
            pltpu.sync_copy(cnt_s.at[pl.ds(r0, RPT)],
                            out_cnt.at[cid, pl.ds(r0, RPT)])

    return pl.kernel(body, out_type=out_type, mesh=mesh,
                     scratch_types=scratch,
                     compiler_params=pltpu.CompilerParams(
                         use_tc_tiling_on_sc=False))


def _tc1_body(x_ref, w_ref, b_ref, a_ref, bo_ref):
    x = x_ref[...]
    w = w_ref[...]
    wa = w[:, :D]
    wb = w[:, D:]
    dn = (((1,), (1,)), ((), ()))
    a_ref[...] = lax.dot_general(x, wa - wb, dn) + b_ref[...]
    bo_ref[...] = lax.dot_general(x, wb, dn)


def _tc2_body(p_ref, c_ref, w_ref, b_ref, a_ref, bo_ref):
    s = p_ref[0] + p_ref[1]
    cnt = c_ref[0, :, 0] + c_ref[1, :, 0]
    hn = s / jnp.maximum(cnt, 1.0)[:, None]
    w = w_ref[...]
    wa = w[:, :H1]
    wb = w[:, H1:]
    dn = (((1,), (1,)), ((), ()))
    a_ref[...] = lax.dot_general(hn, wa - wb, dn) + b_ref[...]
    bo_ref[...] = lax.dot_general(hn, wb, dn)


def _tc3_body(p_ref, c_ref, w3_ref, b3_ref, w4_ref, b4_ref, o_ref):
    s = p_ref[0] + p_ref[1]
    cnt = c_ref[0, :, 0] + c_ref[1, :, 0]
    hn = s / jnp.maximum(cnt, 1.0)[:, None]
    dn = (((1,), (1,)), ((), ()))
    f = jnp.maximum(lax.dot_general(hn, w3_ref[...], dn) + b3_ref[...], 0.0)
    o = jnp.sum(f * w4_ref[...], axis=1, keepdims=True) + b4_ref[0, 0]
    o_ref[...] = jax.nn.sigmoid(o)


def kernel(X, edge_index, W1, b1, W2, b2, W3, b3, W4, b4):
    f32 = jnp.float32
    src1 = edge_index[0].reshape(NW, EPW // 80, 80)
    dst1 = edge_index[1].reshape(NW, EPW // 80, 80)
    src2 = edge_index[0].reshape(NW, EPW // 40, 40)
    dst2 = edge_index[1].reshape(NW, EPW // 40, 40)
    zs1 = jnp.zeros((N, H1), f32)
    zs2 = jnp.zeros((N, H2), f32)
    zc = jnp.zeros((N, LANES), f32)

    grid = (N // BLK,)
    full = lambda shape: pl.BlockSpec(shape, lambda i: (0,) * len(shape))
    rows = lambda w: pl.BlockSpec((BLK, w), lambda i: (i, 0))
    parts = lambda w: pl.BlockSpec((NC, BLK, w), lambda i: (0, i, 0))

    A1, B1 = pl.pallas_call(
        _tc1_body,
        grid=grid,
        in_specs=[rows(D), full((H1, 2 * D)), full((1, H1))],
        out_specs=[rows(H1), rows(H1)],
        out_shape=[jax.ShapeDtypeStruct((N, H1), f32)] * 2,
    )(X, W1, b1.reshape(1, H1))

    P1, CNT = _make_sc_edge(H1, True, 80)(A1, B1, dst1, src1, zs1, zc)

    A2, B2 = pl.pallas_call(
        _tc2_body,
        grid=grid,
        in_specs=[parts(H1), parts(LANES), full((H2, 2 * H1)),
                  full((1, H2))],
        out_specs=[rows(H2), rows(H2)],
        out_shape=[jax.ShapeDtypeStruct((N, H2), f32)] * 2,
    )(P1, CNT, W2, b2.reshape(1, H2))

    (P2,) = _make_sc_edge(H2, False, 40)(A2, B2, dst2, src2, zs2)

    out = pl.pallas_call(
        _tc3_body,
        grid=grid,
        in_specs=[parts(H2), parts(LANES), full((HFC, H2)), full((1, HFC)),
                  full((1, HFC)), full((1, 1))],
        out_specs=[rows(1)],
        out_shape=[jax.ShapeDtypeStruct((N, 1), f32)],
    )(P2, CNT, W3, b3.reshape(1, HFC), W4, b4.reshape(1, 1))[0]

    return out.reshape(N)
